# baseline (speedup 1.0000x reference)
import functools

import jax
import jax.numpy as jnp
from jax import lax
from jax.experimental import pallas as pl
from jax.experimental.pallas import tpu as pltpu
from jax.experimental.pallas import tpu_sc as plsc

N = 10000
E = 320000
F_IN = 128
HEADS = 8
HEAD_DIM = 16
HID = 128

RB = 400
NB = N // RB

NW = 32
EW = E // NW
CSUB = 125
EROWS = E // CSUB
RPW = EW // CSUB
KB = RPW // 8
RPT = N // 16


def _proj_body(x_ref, w_ref, b_ref, as_ref, ad_ref, xp_ref, asrc_ref, adst_ref):
    xb = jnp.dot(x_ref[...], w_ref[...], preferred_element_type=jnp.float32) + b_ref[...]
    xp_ref[...] = xb
    asrc_ref[...] = jnp.dot(xb, as_ref[...], preferred_element_type=jnp.float32,
                            precision=lax.Precision.HIGHEST)
    adst_ref[...] = jnp.dot(xb, ad_ref[...], preferred_element_type=jnp.float32,
                            precision=lax.Precision.HIGHEST)


def _sc_edge_body(asrc_hbm, adst_hbm, xp_hbm, src_hbm, dst_hbm,
                  num_out, den_out,
                  sidx, didx, g1, g2, rows, num_sh, den_sh, sem_g, sem_s):
    c = lax.axis_index("c")
    s = lax.axis_index("s")
    wid = c * 16 + s

    zero16 = jnp.zeros((16,), jnp.float32)

    def zrows_body(i, carry):
        for h in range(8):
            rows[0, i, pl.ds(h * 16, 16)] = zero16
        return carry

    lax.fori_loop(0, CSUB, zrows_body, 0)

    def zg_body(i, carry):
        g1[0, i, :] = zero16
        return carry

    lax.fori_loop(0, CSUB, zg_body, 0)

    for m in range(RPT // CSUB):
        pltpu.sync_copy(rows.at[0],
                        num_sh.at[pl.ds(s * RPT + m * CSUB, CSUB)])
        pltpu.sync_copy(g1.at[0, pl.ds(0, CSUB)],
                        den_sh.at[pl.ds(s * RPT + m * CSUB, CSUB)])
    plsc.subcore_barrier()

    def fire_gathers(kc):
        b = kc % 2
        return [
            pltpu.async_copy(asrc_hbm.at[sidx.at[kc]],
                             g1.at[b, pl.ds(0, CSUB)], sem_g),
            pltpu.async_copy(adst_hbm.at[didx.at[kc]],
                             g2.at[pl.ds(0, CSUB)], sem_g),
            pltpu.async_copy(xp_hbm.at[sidx.at[kc]],
                             rows.at[b], sem_g),
        ]

    def blk_body(kb, carry):
        r0 = wid * RPW + kb * 8
        pltpu.sync_copy(src_hbm.at[pl.ds(r0, 8)], sidx)
        pltpu.sync_copy(dst_hbm.at[pl.ds(r0, 8)], didx)
        gd = fire_gathers(0)
        pairs = [None] * 8
        for kc in range(8):
            b = kc % 2
            for d in gd:
                d.wait()

            def ex_body(e, carry2, _b=b):
                a = g1[_b, e, :] + g2[e, :]
                a = jnp.maximum(a, 0.2 * a)
                t = a * 1.4426950408889634
                m = (t + 12582912.0) - 12582912.0
                n = m.astype(jnp.int32)
                r = (a - m * 0.693359375) - m * (-2.12194440e-4)
                p = 1.0 + r * (1.0 + r * (0.5 + r * (
                    0.16666667 + r * (0.041666668 + r * 0.008333334))))
                g1[_b, e, :] = p * plsc.bitcast((n + 127) << 23, jnp.float32)
                return carry2

            lax.fori_loop(0, CSUB, ex_body, 0)

            if kc < 7:
                if kc > 0:
                    for d in pairs[kc - 1]:
                        d.wait()
                gd = fire_gathers(kc + 1)

            def mul_body(e, carry2, _b=b):
                exv = g1[_b, e, :]
                for h in range(8):
                    rows[_b, e, pl.ds(h * 16, 16)] = (
                        rows[_b, e, pl.ds(h * 16, 16)] * exv[h])
                return carry2

            lax.fori_loop(0, CSUB, mul_body, 0)

            pairs[kc] = [
                pltpu.async_copy(rows.at[b],
                                 num_sh.at[didx.at[kc]], sem_s, add=True),
                pltpu.async_copy(g1.at[b, pl.ds(0, CSUB)],
                                 den_sh.at[didx.at[kc]], sem_s, add=True),
            ]
        for kc in (6, 7):
            for d in pairs[kc]:
                d.wait()
        return carry

    lax.fori_loop(0, KB, blk_body, 0)

    plsc.subcore_barrier()
    pltpu.sync_copy(num_sh.at[pl.ds(s * RPT, RPT)], num_out.at[c, s])
    pltpu.sync_copy(den_sh.at[pl.ds(s * RPT, RPT)], den_out.at[c, s])


_sc_edge = functools.partial(
    pl.kernel,
    mesh=plsc.VectorSubcoreMesh(core_axis_name="c", subcore_axis_name="s"),
    out_type=[
        jax.ShapeDtypeStruct((2, 16, RPT, 128), jnp.float32),
        jax.ShapeDtypeStruct((2, 16, RPT, 16), jnp.float32),
    ],
    scratch_types=[
        pltpu.VMEM((8, CSUB), jnp.int32),
        pltpu.VMEM((8, CSUB), jnp.int32),
        pltpu.VMEM((2, 128, 16), jnp.float32),
        pltpu.VMEM((128, 16), jnp.float32),
        pltpu.VMEM((2, CSUB, 128), jnp.float32),
        pltpu.VMEM_SHARED((N, 128), jnp.float32),
        pltpu.VMEM_SHARED((N, 16), jnp.float32),
        pltpu.SemaphoreType.DMA,
        pltpu.SemaphoreType.DMA,
    ],
    compiler_params=pltpu.CompilerParams(use_tc_tiling_on_sc=False,
                                         needs_layout_passes=False),
)(_sc_edge_body)


def _head_body(n0_ref, n1_ref, d0_ref, d1_ref, exp_ref, wl_ref, bl_ref,
               wc_ref, bc_ref, out_ref, acc_ref):
    i = pl.program_id(0)

    @pl.when(i == 0)
    def _():
        acc_ref[...] = jnp.zeros_like(acc_ref)

    nm = n0_ref[0] + n1_ref[0]
    dn = jnp.dot(d0_ref[0] + d1_ref[0], exp_ref[...],
                 preferred_element_type=jnp.float32,
                 precision=lax.Precision.HIGHEST) + 1e-16
    ob = jnp.maximum(nm / dn, 0.0)
    acc_ref[...] += jnp.sum(ob, axis=0, keepdims=True)

    @pl.when(i == NB - 1)
    def _():
        pooled = acc_ref[...] * (1.0 / N)
        hmid = jnp.maximum(
            jnp.dot(pooled, wl_ref[...], preferred_element_type=jnp.float32) + bl_ref[...], 0.0)
        out_ref[...] = jnp.dot(hmid, wc_ref[...],
                               preferred_element_type=jnp.float32) + bc_ref[...]


def kernel(x, edge_index, W_proj, b_proj, att_src, att_dst, W_sem, b_sem,
           q_sem, W_lin, b_lin, W_cls, b_cls):
    f32 = jnp.float32
    eye_rep = jnp.repeat(jnp.eye(HEADS, dtype=f32), HEAD_DIM, axis=0)
    m_src = eye_rep * att_src.reshape(-1)[:, None]
    m_dst = eye_rep * att_dst.reshape(-1)[:, None]
    as16 = jnp.concatenate([m_src, m_src], axis=1)
    ad16 = jnp.concatenate([m_dst, m_dst], axis=1)

    xp, asrc, adst = pl.pallas_call(
        _proj_body,
        grid=(NB,),
        in_specs=[
            pl.BlockSpec((RB, F_IN), lambda i: (i, 0)),
            pl.BlockSpec((F_IN, HID), lambda i: (0, 0)),
            pl.BlockSpec((1, HID), lambda i: (0, 0)),
            pl.BlockSpec((F_IN, 16), lambda i: (0, 0)),
            pl.BlockSpec((F_IN, 16), lambda i: (0, 0)),
        ],
        out_specs=[
            pl.BlockSpec((RB, HID), lambda i: (i, 0)),
            pl.BlockSpec((RB, 16), lambda i: (i, 0)),
            pl.BlockSpec((RB, 16), lambda i: (i, 0)),
        ],
        out_shape=[
            jax.ShapeDtypeStruct((N, HID), f32),
            jax.ShapeDtypeStruct((N, 16), f32),
            jax.ShapeDtypeStruct((N, 16), f32),
        ],
    )(x, W_proj, b_proj.reshape(1, HID), as16, ad16)

    src2 = edge_index[0].reshape(EROWS, CSUB)
    dst2 = edge_index[1].reshape(EROWS, CSUB)

    num_p, den_p = _sc_edge(asrc, adst, xp, src2, dst2)
    num_p = num_p.reshape(2, N, 128)
    den_p = den_p.reshape(2, N, 16)

    expand = jnp.concatenate(
        [jnp.kron(jnp.eye(HEADS, dtype=f32), jnp.ones((1, HEAD_DIM), f32)),
         jnp.zeros((HEADS, HID), f32)], axis=0)
    wc_pad = jnp.pad(W_cls, ((0, 0), (0, HID - W_cls.shape[1])))
    bc_pad = jnp.pad(b_cls, (0, HID - b_cls.shape[0])).reshape(1, HID)

    logits_pad = pl.pallas_call(
        _head_body,
        grid=(NB,),
        in_specs=[
            pl.BlockSpec((1, RB, 128), lambda i: (0, i, 0)),
            pl.BlockSpec((1, RB, 128), lambda i: (1, i, 0)),
            pl.BlockSpec((1, RB, 16), lambda i: (0, i, 0)),
            pl.BlockSpec((1, RB, 16), lambda i: (1, i, 0)),
            pl.BlockSpec((16, HID), lambda i: (0, 0)),
            pl.BlockSpec((HID, HID), lambda i: (0, 0)),
            pl.BlockSpec((1, HID), lambda i: (0, 0)),
            pl.BlockSpec((HID, HID), lambda i: (0, 0)),
            pl.BlockSpec((1, HID), lambda i: (0, 0)),
        ],
        out_specs=pl.BlockSpec((1, HID), lambda i: (0, 0)),
        out_shape=jax.ShapeDtypeStruct((1, HID), f32),
        scratch_shapes=[pltpu.VMEM((1, HID), f32)],
    )(num_p, num_p, den_p, den_p, expand, W_lin, b_lin.reshape(1, HID),
      wc_pad, bc_pad)

    return logits_pad[0, :2]


# speedup vs baseline: 1.3046x; 1.3046x over previous
import functools

import jax
import jax.numpy as jnp
from jax import lax
from jax.experimental import pallas as pl
from jax.experimental.pallas import tpu as pltpu
from jax.experimental.pallas import tpu_sc as plsc

N = 10000
E = 320000
F_IN = 128
HEADS = 8
HEAD_DIM = 16
HID = 128

RB = 400
NB = N // RB

NW = 32
EW = E // NW
CSUB = 125
EROWS = E // CSUB
RPW = EW // CSUB
KB = RPW // 8
RPT = N // 16


def _proj_body(x_ref, w_ref, b_ref, as_ref, ad_ref, xp_ref, asrc_ref, adst_ref):
    xb = jnp.dot(x_ref[...], w_ref[...], preferred_element_type=jnp.float32) + b_ref[...]
    xp_ref[...] = xb
    asrc_ref[...] = jnp.dot(xb, as_ref[...], preferred_element_type=jnp.float32,
                            precision=lax.Precision.HIGHEST)
    adst_ref[...] = jnp.dot(xb, ad_ref[...], preferred_element_type=jnp.float32,
                            precision=lax.Precision.HIGHEST)


def _sc_edge_body(asrc_hbm, adst_hbm, xp_hbm, src_hbm, dst_hbm,
                  num_out, den_out,
                  sidx, didx, g1, g2, rows, num_sh, den_sh, sem_g, sem_s):
    c = lax.axis_index("c")
    s = lax.axis_index("s")
    wid = c * 16 + s

    zero16 = jnp.zeros((16,), jnp.float32)

    def zrows_body(i, carry):
        for h in range(8):
            rows[0, i, pl.ds(h * 16, 16)] = zero16
        return carry

    lax.fori_loop(0, CSUB, zrows_body, 0)

    def zg_body(i, carry):
        g1[0, i, :] = zero16
        return carry

    lax.fori_loop(0, CSUB, zg_body, 0)

    for m in range(RPT // CSUB):
        pltpu.sync_copy(rows.at[0],
                        num_sh.at[pl.ds(s * RPT + m * CSUB, CSUB)])
        pltpu.sync_copy(g1.at[0, pl.ds(0, CSUB)],
                        den_sh.at[pl.ds(s * RPT + m * CSUB, CSUB)])
    plsc.subcore_barrier()

    def fire_gathers(kc):
        b = kc % 2
        return [
            pltpu.async_copy(asrc_hbm.at[sidx.at[kc]],
                             g1.at[b, pl.ds(0, CSUB)], sem_g),
            pltpu.async_copy(adst_hbm.at[didx.at[kc]],
                             g2.at[pl.ds(0, CSUB)], sem_g),
            pltpu.async_copy(xp_hbm.at[sidx.at[kc]],
                             rows.at[b], sem_g),
        ]

    def blk_body(kb, carry):
        r0 = wid * RPW + kb * 8
        pltpu.sync_copy(src_hbm.at[pl.ds(r0, 8)], sidx)
        pltpu.sync_copy(dst_hbm.at[pl.ds(r0, 8)], didx)
        gd = fire_gathers(0)
        pairs = [None] * 8
        for kc in range(8):
            b = kc % 2
            for d in gd:
                d.wait()

            def ex_body(e, carry2, _b=b):
                a = g1[_b, e, :] + g2[e, :]
                a = jnp.maximum(a, 0.2 * a)
                g1[_b, e, :] = jnp.exp(a)
                return carry2

            lax.fori_loop(0, CSUB, ex_body, 0)

            if kc < 7:
                if kc > 0:
                    for d in pairs[kc - 1]:
                        d.wait()
                gd = fire_gathers(kc + 1)

            def mul_body(e, carry2, _b=b):
                exv = g1[_b, e, :]
                for h in range(8):
                    rows[_b, e, pl.ds(h * 16, 16)] = (
                        rows[_b, e, pl.ds(h * 16, 16)] * exv[h])
                return carry2

            lax.fori_loop(0, CSUB, mul_body, 0)

            pairs[kc] = [
                pltpu.async_copy(rows.at[b],
                                 num_sh.at[didx.at[kc]], sem_s, add=True),
                pltpu.async_copy(g1.at[b, pl.ds(0, CSUB)],
                                 den_sh.at[didx.at[kc]], sem_s, add=True),
            ]
        for kc in (6, 7):
            for d in pairs[kc]:
                d.wait()
        return carry

    lax.fori_loop(0, KB, blk_body, 0)

    plsc.subcore_barrier()
    pltpu.sync_copy(num_sh.at[pl.ds(s * RPT, RPT)], num_out.at[c, s])
    pltpu.sync_copy(den_sh.at[pl.ds(s * RPT, RPT)], den_out.at[c, s])


_sc_edge = functools.partial(
    pl.kernel,
    mesh=plsc.VectorSubcoreMesh(core_axis_name="c", subcore_axis_name="s"),
    out_type=[
        jax.ShapeDtypeStruct((2, 16, RPT, 128), jnp.float32),
        jax.ShapeDtypeStruct((2, 16, RPT, 16), jnp.float32),
    ],
    scratch_types=[
        pltpu.VMEM((8, CSUB), jnp.int32),
        pltpu.VMEM((8, CSUB), jnp.int32),
        pltpu.VMEM((2, 128, 16), jnp.float32),
        pltpu.VMEM((128, 16), jnp.float32),
        pltpu.VMEM((2, CSUB, 128), jnp.float32),
        pltpu.VMEM_SHARED((N, 128), jnp.float32),
        pltpu.VMEM_SHARED((N, 16), jnp.float32),
        pltpu.SemaphoreType.DMA,
        pltpu.SemaphoreType.DMA,
    ],
    compiler_params=pltpu.CompilerParams(use_tc_tiling_on_sc=False,
                                         needs_layout_passes=False),
)(_sc_edge_body)


def _head_body(n0_ref, n1_ref, d0_ref, d1_ref, exp_ref, wl_ref, bl_ref,
               wc_ref, bc_ref, out_ref, acc_ref):
    i = pl.program_id(0)

    @pl.when(i == 0)
    def _():
        acc_ref[...] = jnp.zeros_like(acc_ref)

    nm = n0_ref[0] + n1_ref[0]
    dn = jnp.dot(d0_ref[0] + d1_ref[0], exp_ref[...],
                 preferred_element_type=jnp.float32,
                 precision=lax.Precision.HIGHEST) + 1e-16
    ob = jnp.maximum(nm / dn, 0.0)
    acc_ref[...] += jnp.sum(ob, axis=0, keepdims=True)

    @pl.when(i == NB - 1)
    def _():
        pooled = acc_ref[...] * (1.0 / N)
        hmid = jnp.maximum(
            jnp.dot(pooled, wl_ref[...], preferred_element_type=jnp.float32) + bl_ref[...], 0.0)
        out_ref[...] = jnp.dot(hmid, wc_ref[...],
                               preferred_element_type=jnp.float32) + bc_ref[...]


def kernel(x, edge_index, W_proj, b_proj, att_src, att_dst, W_sem, b_sem,
           q_sem, W_lin, b_lin, W_cls, b_cls):
    f32 = jnp.float32
    eye_rep = jnp.repeat(jnp.eye(HEADS, dtype=f32), HEAD_DIM, axis=0)
    m_src = eye_rep * att_src.reshape(-1)[:, None]
    m_dst = eye_rep * att_dst.reshape(-1)[:, None]
    as16 = jnp.concatenate([m_src, m_src], axis=1)
    ad16 = jnp.concatenate([m_dst, m_dst], axis=1)

    xp, asrc, adst = pl.pallas_call(
        _proj_body,
        grid=(NB,),
        in_specs=[
            pl.BlockSpec((RB, F_IN), lambda i: (i, 0)),
            pl.BlockSpec((F_IN, HID), lambda i: (0, 0)),
            pl.BlockSpec((1, HID), lambda i: (0, 0)),
            pl.BlockSpec((F_IN, 16), lambda i: (0, 0)),
            pl.BlockSpec((F_IN, 16), lambda i: (0, 0)),
        ],
        out_specs=[
            pl.BlockSpec((RB, HID), lambda i: (i, 0)),
            pl.BlockSpec((RB, 16), lambda i: (i, 0)),
            pl.BlockSpec((RB, 16), lambda i: (i, 0)),
        ],
        out_shape=[
            jax.ShapeDtypeStruct((N, HID), f32),
            jax.ShapeDtypeStruct((N, 16), f32),
            jax.ShapeDtypeStruct((N, 16), f32),
        ],
    )(x, W_proj, b_proj.reshape(1, HID), as16, ad16)

    src2 = edge_index[0].reshape(EROWS, CSUB)
    dst2 = edge_index[1].reshape(EROWS, CSUB)

    num_p, den_p = _sc_edge(asrc, adst, xp, src2, dst2)
    num_p = num_p.reshape(2, N, 128)
    den_p = den_p.reshape(2, N, 16)

    expand = jnp.concatenate(
        [jnp.kron(jnp.eye(HEADS, dtype=f32), jnp.ones((1, HEAD_DIM), f32)),
         jnp.zeros((HEADS, HID), f32)], axis=0)
    wc_pad = jnp.pad(W_cls, ((0, 0), (0, HID - W_cls.shape[1])))
    bc_pad = jnp.pad(b_cls, (0, HID - b_cls.shape[0])).reshape(1, HID)

    logits_pad = pl.pallas_call(
        _head_body,
        grid=(NB,),
        in_specs=[
            pl.BlockSpec((1, RB, 128), lambda i: (0, i, 0)),
            pl.BlockSpec((1, RB, 128), lambda i: (1, i, 0)),
            pl.BlockSpec((1, RB, 16), lambda i: (0, i, 0)),
            pl.BlockSpec((1, RB, 16), lambda i: (1, i, 0)),
            pl.BlockSpec((16, HID), lambda i: (0, 0)),
            pl.BlockSpec((HID, HID), lambda i: (0, 0)),
            pl.BlockSpec((1, HID), lambda i: (0, 0)),
            pl.BlockSpec((HID, HID), lambda i: (0, 0)),
            pl.BlockSpec((1, HID), lambda i: (0, 0)),
        ],
        out_specs=pl.BlockSpec((1, HID), lambda i: (0, 0)),
        out_shape=jax.ShapeDtypeStruct((1, HID), f32),
        scratch_shapes=[pltpu.VMEM((1, HID), f32)],
    )(num_p, num_p, den_p, den_p, expand, W_lin, b_lin.reshape(1, HID),
      wc_pad, bc_pad)

    return logits_pad[0, :2]


# speedup vs baseline: 1.3064x; 1.0014x over previous
import functools

import jax
import jax.numpy as jnp
from jax import lax
from jax.experimental import pallas as pl
from jax.experimental.pallas import tpu as pltpu
from jax.experimental.pallas import tpu_sc as plsc

N = 10000
E = 320000
F_IN = 128
HEADS = 8
HEAD_DIM = 16
HID = 128

RB = 400
NB = N // RB

NW = 32
EW = E // NW
CSUB = 125
EROWS = E // CSUB
RPW = EW // CSUB
KB = RPW // 8
RPT = N // 16


def _proj_body(x_ref, w_ref, b_ref, as_ref, ad_ref, xp_ref, asrc_ref, adst_ref):
    xb = jnp.dot(x_ref[...], w_ref[...], preferred_element_type=jnp.float32) + b_ref[...]
    xp_ref[...] = xb
    asrc_ref[...] = jnp.dot(xb, as_ref[...], preferred_element_type=jnp.float32,
                            precision=lax.Precision.HIGHEST)
    adst_ref[...] = jnp.dot(xb, ad_ref[...], preferred_element_type=jnp.float32,
                            precision=lax.Precision.HIGHEST)


def _sc_edge_body(asrc_hbm, adst_hbm, xp_hbm, eidx_hbm,
                  num_out, den_out,
                  eidx, didx_s, g1, g2, rows, num_sh, den_sh, sem_g, sem_s):
    c = lax.axis_index("c")
    s = lax.axis_index("s")
    wid = c * 16 + s

    zero16 = jnp.zeros((16,), jnp.float32)
    zero16i = jnp.zeros((16,), jnp.int32)

    def zrows_body(i, carry):
        for bb in range(2):
            for h in range(8):
                rows[bb, i, pl.ds(h * 16, 16)] = zero16
        return carry

    lax.fori_loop(0, CSUB, zrows_body, 0)

    def zg_body(i, carry):
        g1[0, i, :] = zero16
        g1[1, i, :] = zero16
        return carry

    lax.fori_loop(0, CSUB, zg_body, 0)

    for bb in range(2):
        for i in range(7):
            didx_s[bb, pl.ds(i * 16, 16)] = zero16i
        didx_s[bb, pl.ds(109, 16)] = zero16i

    for m in range(RPT // CSUB):
        pltpu.sync_copy(rows.at[0],
                        num_sh.at[pl.ds(s * RPT + m * CSUB, CSUB)])
        pltpu.sync_copy(g1.at[0, pl.ds(0, CSUB)],
                        den_sh.at[pl.ds(s * RPT + m * CSUB, CSUB)])
    plsc.subcore_barrier()

    for bb in range(2):
        pltpu.async_copy(rows.at[bb],
                         num_sh.at[didx_s.at[bb, pl.ds(0, CSUB)]],
                         sem_s, add=True)
        pltpu.async_copy(g1.at[bb, pl.ds(0, CSUB)],
                         den_sh.at[didx_s.at[bb, pl.ds(0, CSUB)]],
                         sem_s, add=True)

    def fire_gathers(kc):
        b = kc % 2
        return [
            pltpu.async_copy(asrc_hbm.at[eidx.at[kc, 0]],
                             g1.at[b, pl.ds(0, CSUB)], sem_g),
            pltpu.async_copy(adst_hbm.at[eidx.at[kc, 1]],
                             g2.at[pl.ds(0, CSUB)], sem_g),
            pltpu.async_copy(xp_hbm.at[eidx.at[kc, 0]],
                             rows.at[b], sem_g),
        ]

    def drain_pair():
        pltpu.make_async_copy(xp_hbm.at[pl.ds(0, CSUB)],
                              rows.at[0], sem_s).wait()
        pltpu.make_async_copy(asrc_hbm.at[pl.ds(0, CSUB)],
                              g1.at[0, pl.ds(0, CSUB)], sem_s).wait()

    def blk_body(kb, carry):
        r0 = wid * RPW + kb * 8
        pltpu.sync_copy(eidx_hbm.at[pl.ds(r0, 8)], eidx)
        drain_pair()
        gd = fire_gathers(0)
        for kc in range(8):
            b = kc % 2
            for d in gd:
                d.wait()

            for i in range(7):
                didx_s[b, pl.ds(i * 16, 16)] = eidx[kc, 1, pl.ds(i * 16, 16)]
            didx_s[b, pl.ds(109, 16)] = eidx[kc, 1, pl.ds(109, 16)]

            def ex_body(e, carry2, _b=b):
                a = g1[_b, e, :] + g2[e, :]
                a = jnp.maximum(a, 0.2 * a)
                g1[_b, e, :] = jnp.exp(a)
                return carry2

            lax.fori_loop(0, CSUB, ex_body, 0)

            if kc < 7:
                drain_pair()
                gd = fire_gathers(kc + 1)

            def mul_body(e, carry2, _b=b):
                exv = g1[_b, e, :]
                for h in range(8):
                    rows[_b, e, pl.ds(h * 16, 16)] = (
                        rows[_b, e, pl.ds(h * 16, 16)] * exv[h])
                return carry2

            lax.fori_loop(0, CSUB, mul_body, 0)

            pltpu.async_copy(rows.at[b],
                             num_sh.at[didx_s.at[b, pl.ds(0, CSUB)]],
                             sem_s, add=True)
            pltpu.async_copy(g1.at[b, pl.ds(0, CSUB)],
                             den_sh.at[didx_s.at[b, pl.ds(0, CSUB)]],
                             sem_s, add=True)
        return carry

    lax.fori_loop(0, KB, blk_body, 0)

    drain_pair()
    drain_pair()
    plsc.subcore_barrier()
    pltpu.sync_copy(num_sh.at[pl.ds(s * RPT, RPT)],
                    num_out.at[c, pl.ds(s * RPT, RPT)])
    pltpu.sync_copy(den_sh.at[pl.ds(s * RPT, RPT)],
                    den_out.at[c, pl.ds(s * RPT, RPT)])


_sc_edge = functools.partial(
    pl.kernel,
    mesh=plsc.VectorSubcoreMesh(core_axis_name="c", subcore_axis_name="s"),
    out_type=[
        jax.ShapeDtypeStruct((2, N, 128), jnp.float32),
        jax.ShapeDtypeStruct((2, N, 16), jnp.float32),
    ],
    scratch_types=[
        pltpu.VMEM((8, 2, CSUB), jnp.int32),
        pltpu.VMEM((2, CSUB), jnp.int32),
        pltpu.VMEM((2, 128, 16), jnp.float32),
        pltpu.VMEM((128, 16), jnp.float32),
        pltpu.VMEM((2, CSUB, 128), jnp.float32),
        pltpu.VMEM_SHARED((N, 128), jnp.float32),
        pltpu.VMEM_SHARED((N, 16), jnp.float32),
        pltpu.SemaphoreType.DMA,
        pltpu.SemaphoreType.DMA,
    ],
    compiler_params=pltpu.CompilerParams(use_tc_tiling_on_sc=False,
                                         needs_layout_passes=False),
)(_sc_edge_body)


def _head_body(n0_ref, n1_ref, d0_ref, d1_ref, exp_ref, wl_ref, bl_ref,
               wc_ref, bc_ref, out_ref, acc_ref):
    i = pl.program_id(0)

    @pl.when(i == 0)
    def _():
        acc_ref[...] = jnp.zeros_like(acc_ref)

    nm = n0_ref[0] + n1_ref[0]
    dn = jnp.dot(d0_ref[0] + d1_ref[0], exp_ref[...],
                 preferred_element_type=jnp.float32,
                 precision=lax.Precision.HIGHEST) + 1e-16
    ob = jnp.maximum(nm / dn, 0.0)
    acc_ref[...] += jnp.sum(ob, axis=0, keepdims=True)

    @pl.when(i == NB - 1)
    def _():
        pooled = acc_ref[...] * (1.0 / N)
        hmid = jnp.maximum(
            jnp.dot(pooled, wl_ref[...], preferred_element_type=jnp.float32) + bl_ref[...], 0.0)
        out_ref[...] = jnp.dot(hmid, wc_ref[...],
                               preferred_element_type=jnp.float32) + bc_ref[...]


def kernel(x, edge_index, W_proj, b_proj, att_src, att_dst, W_sem, b_sem,
           q_sem, W_lin, b_lin, W_cls, b_cls):
    f32 = jnp.float32
    eye_rep = jnp.repeat(jnp.eye(HEADS, dtype=f32), HEAD_DIM, axis=0)
    m_src = eye_rep * att_src.reshape(-1)[:, None]
    m_dst = eye_rep * att_dst.reshape(-1)[:, None]
    as16 = jnp.concatenate([m_src, m_src], axis=1)
    ad16 = jnp.concatenate([m_dst, m_dst], axis=1)

    xp, asrc, adst = pl.pallas_call(
        _proj_body,
        grid=(NB,),
        in_specs=[
            pl.BlockSpec((RB, F_IN), lambda i: (i, 0)),
            pl.BlockSpec((F_IN, HID), lambda i: (0, 0)),
            pl.BlockSpec((1, HID), lambda i: (0, 0)),
            pl.BlockSpec((F_IN, 16), lambda i: (0, 0)),
            pl.BlockSpec((F_IN, 16), lambda i: (0, 0)),
        ],
        out_specs=[
            pl.BlockSpec((RB, HID), lambda i: (i, 0)),
            pl.BlockSpec((RB, 16), lambda i: (i, 0)),
            pl.BlockSpec((RB, 16), lambda i: (i, 0)),
        ],
        out_shape=[
            jax.ShapeDtypeStruct((N, HID), f32),
            jax.ShapeDtypeStruct((N, 16), f32),
            jax.ShapeDtypeStruct((N, 16), f32),
        ],
    )(x, W_proj, b_proj.reshape(1, HID), as16, ad16)

    eidx_arr = jnp.stack([edge_index[0].reshape(EROWS, CSUB),
                          edge_index[1].reshape(EROWS, CSUB)], axis=1)

    num_p, den_p = _sc_edge(asrc, adst, xp, eidx_arr)

    expand = jnp.concatenate(
        [jnp.kron(jnp.eye(HEADS, dtype=f32), jnp.ones((1, HEAD_DIM), f32)),
         jnp.zeros((HEADS, HID), f32)], axis=0)
    wc_pad = jnp.pad(W_cls, ((0, 0), (0, HID - W_cls.shape[1])))
    bc_pad = jnp.pad(b_cls, (0, HID - b_cls.shape[0])).reshape(1, HID)

    logits_pad = pl.pallas_call(
        _head_body,
        grid=(NB,),
        in_specs=[
            pl.BlockSpec((1, RB, 128), lambda i: (0, i, 0)),
            pl.BlockSpec((1, RB, 128), lambda i: (1, i, 0)),
            pl.BlockSpec((1, RB, 16), lambda i: (0, i, 0)),
            pl.BlockSpec((1, RB, 16), lambda i: (1, i, 0)),
            pl.BlockSpec((16, HID), lambda i: (0, 0)),
            pl.BlockSpec((HID, HID), lambda i: (0, 0)),
            pl.BlockSpec((1, HID), lambda i: (0, 0)),
            pl.BlockSpec((HID, HID), lambda i: (0, 0)),
            pl.BlockSpec((1, HID), lambda i: (0, 0)),
        ],
        out_specs=pl.BlockSpec((1, HID), lambda i: (0, 0)),
        out_shape=jax.ShapeDtypeStruct((1, HID), f32),
        scratch_shapes=[pltpu.VMEM((1, HID), f32)],
    )(num_p, num_p, den_p, den_p, expand, W_lin, b_lin.reshape(1, HID),
      wc_pad, bc_pad)

    return logits_pad[0, :2]


# speedup vs baseline: 1.8190x; 1.3924x over previous
import functools

import jax
import jax.numpy as jnp
from jax import lax
from jax.experimental import pallas as pl
from jax.experimental.pallas import tpu as pltpu
from jax.experimental.pallas import tpu_sc as plsc

N = 10000
E = 320000
F_IN = 128
HEADS = 8
HEAD_DIM = 16
HID = 128

RB = 400
NB = N // RB

NW = 32
EW = E // NW
CSUB = 125
EROWS = E // CSUB
RPW = EW // CSUB
KB = RPW // 8
RPT = N // 16


def _proj_body(x_ref, w_ref, b_ref, as_ref, ad_ref, xp_ref, asrc_ref, adst_ref):
    xb = jnp.dot(x_ref[...], w_ref[...], preferred_element_type=jnp.float32) + b_ref[...]
    xp_ref[...] = xb
    asrc_ref[...] = jnp.dot(xb, as_ref[...], preferred_element_type=jnp.float32,
                            precision=lax.Precision.HIGHEST)
    adst_ref[...] = jnp.dot(xb, ad_ref[...], preferred_element_type=jnp.float32,
                            precision=lax.Precision.HIGHEST)


def _sc_edge_body(asrc_hbm, adst_hbm, xp_hbm, eidx_hbm,
                  num_out, den_out,
                  eidx, didx_s, g1, g2, rows, num_sh, den_sh, sem_g, sem_s):
    c = lax.axis_index("c")
    s = lax.axis_index("s")
    wid = c * 16 + s

    zero16 = jnp.zeros((16,), jnp.float32)
    zero16i = jnp.zeros((16,), jnp.int32)

    def zrows_body(i, carry):
        for bb in range(2):
            for h in range(8):
                rows[bb, i, pl.ds(h * 16, 16)] = zero16
        return carry

    lax.fori_loop(0, CSUB, zrows_body, 0)

    def zg_body(i, carry):
        g1[0, i, :] = zero16
        g1[1, i, :] = zero16
        return carry

    lax.fori_loop(0, CSUB, zg_body, 0)

    for bb in range(2):
        for i in range(7):
            didx_s[bb, pl.ds(i * 16, 16)] = zero16i
        didx_s[bb, pl.ds(109, 16)] = zero16i

    for m in range(RPT // CSUB):
        pltpu.sync_copy(rows.at[0],
                        num_sh.at[pl.ds(s * RPT + m * CSUB, CSUB)])
        pltpu.sync_copy(g1.at[0, pl.ds(0, CSUB)],
                        den_sh.at[pl.ds(s * RPT + m * CSUB, CSUB)])
    plsc.subcore_barrier()

    for bb in range(2):
        pltpu.async_copy(rows.at[bb],
                         num_sh.at[didx_s.at[bb, pl.ds(0, CSUB)]],
                         sem_s, add=True)
        pltpu.async_copy(g1.at[bb, pl.ds(0, CSUB)],
                         den_sh.at[didx_s.at[bb, pl.ds(0, CSUB)]],
                         sem_s, add=True)

    def fire_gathers(kc):
        b = kc % 2
        return [
            pltpu.async_copy(asrc_hbm.at[eidx.at[kc, 0]],
                             g1.at[b, pl.ds(0, CSUB)], sem_g),
            pltpu.async_copy(adst_hbm.at[eidx.at[kc, 1]],
                             g2.at[pl.ds(0, CSUB)], sem_g),
            pltpu.async_copy(xp_hbm.at[eidx.at[kc, 0]],
                             rows.at[b], sem_g),
        ]

    def drain_pair():
        pltpu.make_async_copy(xp_hbm.at[pl.ds(0, CSUB)],
                              rows.at[0], sem_s).wait()
        pltpu.make_async_copy(asrc_hbm.at[pl.ds(0, CSUB)],
                              g1.at[0, pl.ds(0, CSUB)], sem_s).wait()

    def blk_body(kb, carry):
        r0 = wid * RPW + kb * 8
        pltpu.sync_copy(eidx_hbm.at[pl.ds(r0, 8)], eidx)
        drain_pair()
        gd = fire_gathers(0)
        for kc in range(8):
            b = kc % 2
            for d in gd:
                d.wait()

            for i in range(7):
                didx_s[b, pl.ds(i * 16, 16)] = eidx[kc, 1, pl.ds(i * 16, 16)]
            didx_s[b, pl.ds(109, 16)] = eidx[kc, 1, pl.ds(109, 16)]

            @plsc.parallel_loop(0, CSUB, unroll=2)
            def _(e, _b=b):
                a = g1[_b, e, :] + g2[e, :]
                a = jnp.maximum(a, 0.2 * a)
                g1[_b, e, :] = jnp.exp(a)

            if kc < 7:
                drain_pair()
                gd = fire_gathers(kc + 1)

            @plsc.parallel_loop(0, CSUB, unroll=2)
            def _(e, _b=b):
                exv = g1[_b, e, :]
                for h in range(8):
                    rows[_b, e, pl.ds(h * 16, 16)] = (
                        rows[_b, e, pl.ds(h * 16, 16)] * exv[h])

            pltpu.async_copy(rows.at[b],
                             num_sh.at[didx_s.at[b, pl.ds(0, CSUB)]],
                             sem_s, add=True)
            pltpu.async_copy(g1.at[b, pl.ds(0, CSUB)],
                             den_sh.at[didx_s.at[b, pl.ds(0, CSUB)]],
                             sem_s, add=True)
        return carry

    lax.fori_loop(0, KB, blk_body, 0)

    drain_pair()
    drain_pair()
    plsc.subcore_barrier()
    pltpu.sync_copy(num_sh.at[pl.ds(s * RPT, RPT)],
                    num_out.at[c, pl.ds(s * RPT, RPT)])
    pltpu.sync_copy(den_sh.at[pl.ds(s * RPT, RPT)],
                    den_out.at[c, pl.ds(s * RPT, RPT)])


_sc_edge = functools.partial(
    pl.kernel,
    mesh=plsc.VectorSubcoreMesh(core_axis_name="c", subcore_axis_name="s"),
    out_type=[
        jax.ShapeDtypeStruct((2, N, 128), jnp.float32),
        jax.ShapeDtypeStruct((2, N, 16), jnp.float32),
    ],
    scratch_types=[
        pltpu.VMEM((8, 2, CSUB), jnp.int32),
        pltpu.VMEM((2, CSUB), jnp.int32),
        pltpu.VMEM((2, 128, 16), jnp.float32),
        pltpu.VMEM((128, 16), jnp.float32),
        pltpu.VMEM((2, CSUB, 128), jnp.float32),
        pltpu.VMEM_SHARED((N, 128), jnp.float32),
        pltpu.VMEM_SHARED((N, 16), jnp.float32),
        pltpu.SemaphoreType.DMA,
        pltpu.SemaphoreType.DMA,
    ],
    compiler_params=pltpu.CompilerParams(use_tc_tiling_on_sc=False,
                                         needs_layout_passes=False),
)(_sc_edge_body)


def _head_body(n0_ref, n1_ref, d0_ref, d1_ref, exp_ref, wl_ref, bl_ref,
               wc_ref, bc_ref, out_ref, acc_ref):
    i = pl.program_id(0)

    @pl.when(i == 0)
    def _():
        acc_ref[...] = jnp.zeros_like(acc_ref)

    nm = n0_ref[0] + n1_ref[0]
    dn = jnp.dot(d0_ref[0] + d1_ref[0], exp_ref[...],
                 preferred_element_type=jnp.float32,
                 precision=lax.Precision.HIGHEST) + 1e-16
    ob = jnp.maximum(nm / dn, 0.0)
    acc_ref[...] += jnp.sum(ob, axis=0, keepdims=True)

    @pl.when(i == NB - 1)
    def _():
        pooled = acc_ref[...] * (1.0 / N)
        hmid = jnp.maximum(
            jnp.dot(pooled, wl_ref[...], preferred_element_type=jnp.float32) + bl_ref[...], 0.0)
        out_ref[...] = jnp.dot(hmid, wc_ref[...],
                               preferred_element_type=jnp.float32) + bc_ref[...]


def kernel(x, edge_index, W_proj, b_proj, att_src, att_dst, W_sem, b_sem,
           q_sem, W_lin, b_lin, W_cls, b_cls):
    f32 = jnp.float32
    eye_rep = jnp.repeat(jnp.eye(HEADS, dtype=f32), HEAD_DIM, axis=0)
    m_src = eye_rep * att_src.reshape(-1)[:, None]
    m_dst = eye_rep * att_dst.reshape(-1)[:, None]
    as16 = jnp.concatenate([m_src, m_src], axis=1)
    ad16 = jnp.concatenate([m_dst, m_dst], axis=1)

    xp, asrc, adst = pl.pallas_call(
        _proj_body,
        grid=(NB,),
        in_specs=[
            pl.BlockSpec((RB, F_IN), lambda i: (i, 0)),
            pl.BlockSpec((F_IN, HID), lambda i: (0, 0)),
            pl.BlockSpec((1, HID), lambda i: (0, 0)),
            pl.BlockSpec((F_IN, 16), lambda i: (0, 0)),
            pl.BlockSpec((F_IN, 16), lambda i: (0, 0)),
        ],
        out_specs=[
            pl.BlockSpec((RB, HID), lambda i: (i, 0)),
            pl.BlockSpec((RB, 16), lambda i: (i, 0)),
            pl.BlockSpec((RB, 16), lambda i: (i, 0)),
        ],
        out_shape=[
            jax.ShapeDtypeStruct((N, HID), f32),
            jax.ShapeDtypeStruct((N, 16), f32),
            jax.ShapeDtypeStruct((N, 16), f32),
        ],
    )(x, W_proj, b_proj.reshape(1, HID), as16, ad16)

    eidx_arr = jnp.stack([edge_index[0].reshape(EROWS, CSUB),
                          edge_index[1].reshape(EROWS, CSUB)], axis=1)

    num_p, den_p = _sc_edge(asrc, adst, xp, eidx_arr)

    expand = jnp.concatenate(
        [jnp.kron(jnp.eye(HEADS, dtype=f32), jnp.ones((1, HEAD_DIM), f32)),
         jnp.zeros((HEADS, HID), f32)], axis=0)
    wc_pad = jnp.pad(W_cls, ((0, 0), (0, HID - W_cls.shape[1])))
    bc_pad = jnp.pad(b_cls, (0, HID - b_cls.shape[0])).reshape(1, HID)

    logits_pad = pl.pallas_call(
        _head_body,
        grid=(NB,),
        in_specs=[
            pl.BlockSpec((1, RB, 128), lambda i: (0, i, 0)),
            pl.BlockSpec((1, RB, 128), lambda i: (1, i, 0)),
            pl.BlockSpec((1, RB, 16), lambda i: (0, i, 0)),
            pl.BlockSpec((1, RB, 16), lambda i: (1, i, 0)),
            pl.BlockSpec((16, HID), lambda i: (0, 0)),
            pl.BlockSpec((HID, HID), lambda i: (0, 0)),
            pl.BlockSpec((1, HID), lambda i: (0, 0)),
            pl.BlockSpec((HID, HID), lambda i: (0, 0)),
            pl.BlockSpec((1, HID), lambda i: (0, 0)),
        ],
        out_specs=pl.BlockSpec((1, HID), lambda i: (0, 0)),
        out_shape=jax.ShapeDtypeStruct((1, HID), f32),
        scratch_shapes=[pltpu.VMEM((1, HID), f32)],
    )(num_p, num_p, den_p, den_p, expand, W_lin, b_lin.reshape(1, HID),
      wc_pad, bc_pad)

    return logits_pad[0, :2]


# speedup vs baseline: 1.8505x; 1.0173x over previous
import functools

import jax
import jax.numpy as jnp
from jax import lax
from jax.experimental import pallas as pl
from jax.experimental.pallas import tpu as pltpu
from jax.experimental.pallas import tpu_sc as plsc

N = 10000
E = 320000
F_IN = 128
HEADS = 8
HEAD_DIM = 16
HID = 128

RB = 400
NB = N // RB

NW = 32
EW = E // NW
CSUB = 125
EROWS = E // CSUB
RPW = EW // CSUB
KB = RPW // 8
RPT = N // 16


def _proj_body(x_ref, w_ref, b_ref, as_ref, ad_ref, xp_ref, asrc_ref, adst_ref):
    xb = jnp.dot(x_ref[...], w_ref[...], preferred_element_type=jnp.float32) + b_ref[...]
    xp_ref[...] = xb
    asrc_ref[...] = jnp.dot(xb, as_ref[...], preferred_element_type=jnp.float32,
                            precision=lax.Precision.HIGHEST)
    adst_ref[...] = jnp.dot(xb, ad_ref[...], preferred_element_type=jnp.float32,
                            precision=lax.Precision.HIGHEST)


def _sc_edge_body(asrc_hbm, adst_hbm, xp_hbm, eidx_hbm,
                  num_out, den_out,
                  eidx, didx_s, g1, g2, rows, num_sh, den_sh, sem_g, sem_s):
    c = lax.axis_index("c")
    s = lax.axis_index("s")
    wid = c * 16 + s

    zero16 = jnp.zeros((16,), jnp.float32)
    zero16i = jnp.zeros((16,), jnp.int32)

    def zrows_body(i, carry):
        for bb in range(2):
            for h in range(8):
                rows[bb, i, pl.ds(h * 16, 16)] = zero16
        return carry

    lax.fori_loop(0, CSUB, zrows_body, 0)

    def zg_body(i, carry):
        g1[0, i, :] = zero16
        g1[1, i, :] = zero16
        return carry

    lax.fori_loop(0, CSUB, zg_body, 0)

    for bb in range(2):
        for i in range(7):
            didx_s[bb, pl.ds(i * 16, 16)] = zero16i
        didx_s[bb, pl.ds(109, 16)] = zero16i

    for m in range(RPT // CSUB):
        pltpu.sync_copy(rows.at[0],
                        num_sh.at[pl.ds(s * RPT + m * CSUB, CSUB)])
        pltpu.sync_copy(g1.at[0, pl.ds(0, CSUB)],
                        den_sh.at[pl.ds(s * RPT + m * CSUB, CSUB)])
    plsc.subcore_barrier()

    for bb in range(2):
        pltpu.async_copy(rows.at[bb],
                         num_sh.at[didx_s.at[bb, pl.ds(0, CSUB)]],
                         sem_s, add=True)
        pltpu.async_copy(g1.at[bb, pl.ds(0, CSUB)],
                         den_sh.at[didx_s.at[bb, pl.ds(0, CSUB)]],
                         sem_s, add=True)

    def fire_gathers(kc):
        b = kc % 2
        return [
            pltpu.async_copy(asrc_hbm.at[eidx.at[kc, 0]],
                             g1.at[b, pl.ds(0, CSUB)], sem_g),
            pltpu.async_copy(adst_hbm.at[eidx.at[kc, 1]],
                             g2.at[pl.ds(0, CSUB)], sem_g),
            pltpu.async_copy(xp_hbm.at[eidx.at[kc, 0]],
                             rows.at[b], sem_g),
        ]

    def drain_pair():
        pltpu.make_async_copy(xp_hbm.at[pl.ds(0, CSUB)],
                              rows.at[0], sem_s).wait()
        pltpu.make_async_copy(asrc_hbm.at[pl.ds(0, CSUB)],
                              g1.at[0, pl.ds(0, CSUB)], sem_s).wait()

    def blk_body(kb, carry):
        r0 = wid * RPW + kb * 8
        pltpu.sync_copy(eidx_hbm.at[pl.ds(r0, 8)], eidx)
        drain_pair()
        gd = fire_gathers(0)
        for kc in range(8):
            b = kc % 2
            for d in gd:
                d.wait()

            for i in range(7):
                didx_s[b, pl.ds(i * 16, 16)] = eidx[kc, 1, pl.ds(i * 16, 16)]
            didx_s[b, pl.ds(109, 16)] = eidx[kc, 1, pl.ds(109, 16)]

            @plsc.parallel_loop(0, CSUB, unroll=4)
            def _(e, _b=b):
                a = g1[_b, e, :] + g2[e, :]
                a = jnp.maximum(a, 0.2 * a)
                g1[_b, e, :] = jnp.exp(a)

            if kc < 7:
                drain_pair()
                gd = fire_gathers(kc + 1)

            @plsc.parallel_loop(0, CSUB, unroll=4)
            def _(e, _b=b):
                exv = g1[_b, e, :]
                for h in range(8):
                    rows[_b, e, pl.ds(h * 16, 16)] = (
                        rows[_b, e, pl.ds(h * 16, 16)] * exv[h])

            pltpu.async_copy(rows.at[b],
                             num_sh.at[didx_s.at[b, pl.ds(0, CSUB)]],
                             sem_s, add=True)
            pltpu.async_copy(g1.at[b, pl.ds(0, CSUB)],
                             den_sh.at[didx_s.at[b, pl.ds(0, CSUB)]],
                             sem_s, add=True)
        return carry

    lax.fori_loop(0, KB, blk_body, 0)

    drain_pair()
    drain_pair()
    plsc.subcore_barrier()
    pltpu.sync_copy(num_sh.at[pl.ds(s * RPT, RPT)],
                    num_out.at[c, pl.ds(s * RPT, RPT)])
    pltpu.sync_copy(den_sh.at[pl.ds(s * RPT, RPT)],
                    den_out.at[c, pl.ds(s * RPT, RPT)])


_sc_edge = functools.partial(
    pl.kernel,
    mesh=plsc.VectorSubcoreMesh(core_axis_name="c", subcore_axis_name="s"),
    out_type=[
        jax.ShapeDtypeStruct((2, N, 128), jnp.float32),
        jax.ShapeDtypeStruct((2, N, 16), jnp.float32),
    ],
    scratch_types=[
        pltpu.VMEM((8, 2, CSUB), jnp.int32),
        pltpu.VMEM((2, CSUB), jnp.int32),
        pltpu.VMEM((2, 128, 16), jnp.float32),
        pltpu.VMEM((128, 16), jnp.float32),
        pltpu.VMEM((2, CSUB, 128), jnp.float32),
        pltpu.VMEM_SHARED((N, 128), jnp.float32),
        pltpu.VMEM_SHARED((N, 16), jnp.float32),
        pltpu.SemaphoreType.DMA,
        pltpu.SemaphoreType.DMA,
    ],
    compiler_params=pltpu.CompilerParams(use_tc_tiling_on_sc=False,
                                         needs_layout_passes=False),
)(_sc_edge_body)


def _head_body(n0_ref, n1_ref, d0_ref, d1_ref, exp_ref, wl_ref, bl_ref,
               wc_ref, bc_ref, out_ref, acc_ref):
    i = pl.program_id(0)

    @pl.when(i == 0)
    def _():
        acc_ref[...] = jnp.zeros_like(acc_ref)

    nm = n0_ref[0] + n1_ref[0]
    dn = jnp.dot(d0_ref[0] + d1_ref[0], exp_ref[...],
                 preferred_element_type=jnp.float32,
                 precision=lax.Precision.HIGHEST) + 1e-16
    ob = jnp.maximum(nm / dn, 0.0)
    acc_ref[...] += jnp.sum(ob, axis=0, keepdims=True)

    @pl.when(i == NB - 1)
    def _():
        pooled = acc_ref[...] * (1.0 / N)
        hmid = jnp.maximum(
            jnp.dot(pooled, wl_ref[...], preferred_element_type=jnp.float32) + bl_ref[...], 0.0)
        out_ref[...] = jnp.dot(hmid, wc_ref[...],
                               preferred_element_type=jnp.float32) + bc_ref[...]


def kernel(x, edge_index, W_proj, b_proj, att_src, att_dst, W_sem, b_sem,
           q_sem, W_lin, b_lin, W_cls, b_cls):
    f32 = jnp.float32
    eye_rep = jnp.repeat(jnp.eye(HEADS, dtype=f32), HEAD_DIM, axis=0)
    m_src = eye_rep * att_src.reshape(-1)[:, None]
    m_dst = eye_rep * att_dst.reshape(-1)[:, None]
    as16 = jnp.concatenate([m_src, m_src], axis=1)
    ad16 = jnp.concatenate([m_dst, m_dst], axis=1)

    xp, asrc, adst = pl.pallas_call(
        _proj_body,
        grid=(NB,),
        in_specs=[
            pl.BlockSpec((RB, F_IN), lambda i: (i, 0)),
            pl.BlockSpec((F_IN, HID), lambda i: (0, 0)),
            pl.BlockSpec((1, HID), lambda i: (0, 0)),
            pl.BlockSpec((F_IN, 16), lambda i: (0, 0)),
            pl.BlockSpec((F_IN, 16), lambda i: (0, 0)),
        ],
        out_specs=[
            pl.BlockSpec((RB, HID), lambda i: (i, 0)),
            pl.BlockSpec((RB, 16), lambda i: (i, 0)),
            pl.BlockSpec((RB, 16), lambda i: (i, 0)),
        ],
        out_shape=[
            jax.ShapeDtypeStruct((N, HID), f32),
            jax.ShapeDtypeStruct((N, 16), f32),
            jax.ShapeDtypeStruct((N, 16), f32),
        ],
    )(x, W_proj, b_proj.reshape(1, HID), as16, ad16)

    eidx_arr = jnp.stack([edge_index[0].reshape(EROWS, CSUB),
                          edge_index[1].reshape(EROWS, CSUB)], axis=1)

    num_p, den_p = _sc_edge(asrc, adst, xp, eidx_arr)

    expand = jnp.concatenate(
        [jnp.kron(jnp.eye(HEADS, dtype=f32), jnp.ones((1, HEAD_DIM), f32)),
         jnp.zeros((HEADS, HID), f32)], axis=0)
    wc_pad = jnp.pad(W_cls, ((0, 0), (0, HID - W_cls.shape[1])))
    bc_pad = jnp.pad(b_cls, (0, HID - b_cls.shape[0])).reshape(1, HID)

    logits_pad = pl.pallas_call(
        _head_body,
        grid=(NB,),
        in_specs=[
            pl.BlockSpec((1, RB, 128), lambda i: (0, i, 0)),
            pl.BlockSpec((1, RB, 128), lambda i: (1, i, 0)),
            pl.BlockSpec((1, RB, 16), lambda i: (0, i, 0)),
            pl.BlockSpec((1, RB, 16), lambda i: (1, i, 0)),
            pl.BlockSpec((16, HID), lambda i: (0, 0)),
            pl.BlockSpec((HID, HID), lambda i: (0, 0)),
            pl.BlockSpec((1, HID), lambda i: (0, 0)),
            pl.BlockSpec((HID, HID), lambda i: (0, 0)),
            pl.BlockSpec((1, HID), lambda i: (0, 0)),
        ],
        out_specs=pl.BlockSpec((1, HID), lambda i: (0, 0)),
        out_shape=jax.ShapeDtypeStruct((1, HID), f32),
        scratch_shapes=[pltpu.VMEM((1, HID), f32)],
    )(num_p, num_p, den_p, den_p, expand, W_lin, b_lin.reshape(1, HID),
      wc_pad, bc_pad)

    return logits_pad[0, :2]


# speedup vs baseline: 1.9014x; 1.0275x over previous
import functools

import jax
import jax.numpy as jnp
from jax import lax
from jax.experimental import pallas as pl
from jax.experimental.pallas import tpu as pltpu
from jax.experimental.pallas import tpu_sc as plsc

N = 10000
E = 320000
F_IN = 128
HEADS = 8
HEAD_DIM = 16
HID = 128

RB = 1000
NB = N // RB

NW = 32
EW = E // NW
CSUB = 125
EROWS = E // CSUB
RPW = EW // CSUB
KB = RPW // 8
RPT = N // 16


def _proj_body(x_ref, w_ref, b_ref, as_ref, ad_ref, xp_ref, asrc_ref, adst_ref):
    xb = jnp.dot(x_ref[...], w_ref[...], preferred_element_type=jnp.float32) + b_ref[...]
    xp_ref[...] = xb
    asrc_ref[...] = jnp.dot(xb, as_ref[...], preferred_element_type=jnp.float32,
                            precision=lax.Precision.HIGHEST)
    adst_ref[...] = jnp.dot(xb, ad_ref[...], preferred_element_type=jnp.float32,
                            precision=lax.Precision.HIGHEST)


def _sc_edge_body(asrc_hbm, adst_hbm, xp_hbm, eidx_hbm,
                  num_out, den_out,
                  eidx, didx_s, g1, g2, rows, num_sh, den_sh, sem_g, sem_s):
    c = lax.axis_index("c")
    s = lax.axis_index("s")
    wid = c * 16 + s

    zero16 = jnp.zeros((16,), jnp.float32)
    zero16i = jnp.zeros((16,), jnp.int32)

    def zrows_body(i, carry):
        for bb in range(2):
            for h in range(8):
                rows[bb, i, pl.ds(h * 16, 16)] = zero16
        return carry

    lax.fori_loop(0, CSUB, zrows_body, 0)

    def zg_body(i, carry):
        g1[0, i, :] = zero16
        g1[1, i, :] = zero16
        return carry

    lax.fori_loop(0, CSUB, zg_body, 0)

    for bb in range(2):
        for i in range(7):
            didx_s[bb, pl.ds(i * 16, 16)] = zero16i
        didx_s[bb, pl.ds(109, 16)] = zero16i

    for m in range(RPT // CSUB):
        pltpu.sync_copy(rows.at[0],
                        num_sh.at[pl.ds(s * RPT + m * CSUB, CSUB)])
        pltpu.sync_copy(g1.at[0, pl.ds(0, CSUB)],
                        den_sh.at[pl.ds(s * RPT + m * CSUB, CSUB)])
    plsc.subcore_barrier()

    for bb in range(2):
        pltpu.async_copy(rows.at[bb],
                         num_sh.at[didx_s.at[bb, pl.ds(0, CSUB)]],
                         sem_s, add=True)
        pltpu.async_copy(g1.at[bb, pl.ds(0, CSUB)],
                         den_sh.at[didx_s.at[bb, pl.ds(0, CSUB)]],
                         sem_s, add=True)

    def fire_gathers(kc):
        b = kc % 2
        return [
            pltpu.async_copy(asrc_hbm.at[eidx.at[kc, 0]],
                             g1.at[b, pl.ds(0, CSUB)], sem_g),
            pltpu.async_copy(adst_hbm.at[eidx.at[kc, 1]],
                             g2.at[pl.ds(0, CSUB)], sem_g),
            pltpu.async_copy(xp_hbm.at[eidx.at[kc, 0]],
                             rows.at[b], sem_g),
        ]

    def drain_pair():
        pltpu.make_async_copy(xp_hbm.at[pl.ds(0, CSUB)],
                              rows.at[0], sem_s).wait()
        pltpu.make_async_copy(asrc_hbm.at[pl.ds(0, CSUB)],
                              g1.at[0, pl.ds(0, CSUB)], sem_s).wait()

    def blk_body(kb, carry):
        r0 = wid * RPW + kb * 8
        pltpu.sync_copy(eidx_hbm.at[pl.ds(r0, 8)], eidx)
        drain_pair()
        gd = fire_gathers(0)
        for kc in range(8):
            b = kc % 2
            for d in gd:
                d.wait()

            for i in range(7):
                didx_s[b, pl.ds(i * 16, 16)] = eidx[kc, 1, pl.ds(i * 16, 16)]
            didx_s[b, pl.ds(109, 16)] = eidx[kc, 1, pl.ds(109, 16)]

            @plsc.parallel_loop(0, CSUB, unroll=4)
            def _(e, _b=b):
                a = g1[_b, e, :] + g2[e, :]
                a = jnp.maximum(a, 0.2 * a)
                g1[_b, e, :] = jnp.exp(a)

            if kc < 7:
                drain_pair()
                gd = fire_gathers(kc + 1)

            @plsc.parallel_loop(0, CSUB, unroll=4)
            def _(e, _b=b):
                exv = g1[_b, e, :]
                for h in range(8):
                    rows[_b, e, pl.ds(h * 16, 16)] = (
                        rows[_b, e, pl.ds(h * 16, 16)] * exv[h])

            pltpu.async_copy(rows.at[b],
                             num_sh.at[didx_s.at[b, pl.ds(0, CSUB)]],
                             sem_s, add=True)
            pltpu.async_copy(g1.at[b, pl.ds(0, CSUB)],
                             den_sh.at[didx_s.at[b, pl.ds(0, CSUB)]],
                             sem_s, add=True)
        return carry

    lax.fori_loop(0, KB, blk_body, 0)

    drain_pair()
    drain_pair()
    plsc.subcore_barrier()
    pltpu.sync_copy(num_sh.at[pl.ds(s * RPT, RPT)],
                    num_out.at[c, pl.ds(s * RPT, RPT)])
    pltpu.sync_copy(den_sh.at[pl.ds(s * RPT, RPT)],
                    den_out.at[c, pl.ds(s * RPT, RPT)])


_sc_edge = functools.partial(
    pl.kernel,
    mesh=plsc.VectorSubcoreMesh(core_axis_name="c", subcore_axis_name="s"),
    out_type=[
        jax.ShapeDtypeStruct((2, N, 128), jnp.float32),
        jax.ShapeDtypeStruct((2, N, 16), jnp.float32),
    ],
    scratch_types=[
        pltpu.VMEM((8, 2, CSUB), jnp.int32),
        pltpu.VMEM((2, CSUB), jnp.int32),
        pltpu.VMEM((2, 128, 16), jnp.float32),
        pltpu.VMEM((128, 16), jnp.float32),
        pltpu.VMEM((2, CSUB, 128), jnp.float32),
        pltpu.VMEM_SHARED((N, 128), jnp.float32),
        pltpu.VMEM_SHARED((N, 16), jnp.float32),
        pltpu.SemaphoreType.DMA,
        pltpu.SemaphoreType.DMA,
    ],
    compiler_params=pltpu.CompilerParams(use_tc_tiling_on_sc=False,
                                         needs_layout_passes=False),
)(_sc_edge_body)


def _head_body(n0_ref, n1_ref, d0_ref, d1_ref, exp_ref, wl_ref, bl_ref,
               wc_ref, bc_ref, out_ref, acc_ref):
    i = pl.program_id(0)

    @pl.when(i == 0)
    def _():
        acc_ref[...] = jnp.zeros_like(acc_ref)

    nm = n0_ref[0] + n1_ref[0]
    dn = jnp.dot(d0_ref[0] + d1_ref[0], exp_ref[...],
                 preferred_element_type=jnp.float32,
                 precision=lax.Precision.HIGHEST) + 1e-16
    ob = jnp.maximum(nm / dn, 0.0)
    acc_ref[...] += jnp.sum(ob, axis=0, keepdims=True)

    @pl.when(i == NB - 1)
    def _():
        pooled = acc_ref[...] * (1.0 / N)
        hmid = jnp.maximum(
            jnp.dot(pooled, wl_ref[...], preferred_element_type=jnp.float32) + bl_ref[...], 0.0)
        out_ref[...] = jnp.dot(hmid, wc_ref[...],
                               preferred_element_type=jnp.float32) + bc_ref[...]


def kernel(x, edge_index, W_proj, b_proj, att_src, att_dst, W_sem, b_sem,
           q_sem, W_lin, b_lin, W_cls, b_cls):
    f32 = jnp.float32
    eye_rep = jnp.repeat(jnp.eye(HEADS, dtype=f32), HEAD_DIM, axis=0)
    m_src = eye_rep * att_src.reshape(-1)[:, None]
    m_dst = eye_rep * att_dst.reshape(-1)[:, None]
    as16 = jnp.concatenate([m_src, m_src], axis=1)
    ad16 = jnp.concatenate([m_dst, m_dst], axis=1)

    xp, asrc, adst = pl.pallas_call(
        _proj_body,
        grid=(NB,),
        in_specs=[
            pl.BlockSpec((RB, F_IN), lambda i: (i, 0)),
            pl.BlockSpec((F_IN, HID), lambda i: (0, 0)),
            pl.BlockSpec((1, HID), lambda i: (0, 0)),
            pl.BlockSpec((F_IN, 16), lambda i: (0, 0)),
            pl.BlockSpec((F_IN, 16), lambda i: (0, 0)),
        ],
        out_specs=[
            pl.BlockSpec((RB, HID), lambda i: (i, 0)),
            pl.BlockSpec((RB, 16), lambda i: (i, 0)),
            pl.BlockSpec((RB, 16), lambda i: (i, 0)),
        ],
        out_shape=[
            jax.ShapeDtypeStruct((N, HID), f32),
            jax.ShapeDtypeStruct((N, 16), f32),
            jax.ShapeDtypeStruct((N, 16), f32),
        ],
    )(x, W_proj, b_proj.reshape(1, HID), as16, ad16)

    eidx_arr = jnp.stack([edge_index[0].reshape(EROWS, CSUB),
                          edge_index[1].reshape(EROWS, CSUB)], axis=1)

    num_p, den_p = _sc_edge(asrc, adst, xp, eidx_arr)

    expand = jnp.concatenate(
        [jnp.kron(jnp.eye(HEADS, dtype=f32), jnp.ones((1, HEAD_DIM), f32)),
         jnp.zeros((HEADS, HID), f32)], axis=0)
    wc_pad = jnp.pad(W_cls, ((0, 0), (0, HID - W_cls.shape[1])))
    bc_pad = jnp.pad(b_cls, (0, HID - b_cls.shape[0])).reshape(1, HID)

    logits_pad = pl.pallas_call(
        _head_body,
        grid=(NB,),
        in_specs=[
            pl.BlockSpec((1, RB, 128), lambda i: (0, i, 0)),
            pl.BlockSpec((1, RB, 128), lambda i: (1, i, 0)),
            pl.BlockSpec((1, RB, 16), lambda i: (0, i, 0)),
            pl.BlockSpec((1, RB, 16), lambda i: (1, i, 0)),
            pl.BlockSpec((16, HID), lambda i: (0, 0)),
            pl.BlockSpec((HID, HID), lambda i: (0, 0)),
            pl.BlockSpec((1, HID), lambda i: (0, 0)),
            pl.BlockSpec((HID, HID), lambda i: (0, 0)),
            pl.BlockSpec((1, HID), lambda i: (0, 0)),
        ],
        out_specs=pl.BlockSpec((1, HID), lambda i: (0, 0)),
        out_shape=jax.ShapeDtypeStruct((1, HID), f32),
        scratch_shapes=[pltpu.VMEM((1, HID), f32)],
    )(num_p, num_p, den_p, den_p, expand, W_lin, b_lin.reshape(1, HID),
      wc_pad, bc_pad)

    return logits_pad[0, :2]
